# baseline (speedup 1.0000x reference)
import functools

import jax
import jax.numpy as jnp
from jax import lax
from jax.experimental import pallas as pl
from jax.experimental.pallas import tpu as pltpu
from jax.experimental.pallas import tpu_sc as plsc

N_NODES = 10000
N_EDGES = 320000
N_EMBEDDING = 128
N_DISTANCE = 100
N_HIDDEN = 128

NC = 2
NS = 16
NW = NC * NS

NCH = 2
EDGE_BLK = 1280
REAL_C = N_EDGES // NCH
BLKS_C = REAL_C // EDGE_BLK
W_ROWS = 40
W_EDGES = W_ROWS * 128
PAD_C = NW * W_EDGES
CROWS = PAD_C // 128

N_ACC = 10240
DUMMY_ROW = N_ACC - 1

NODE_BLK = 2000


def _afh_body(a_ref, w_ref, b_ref, o_ref):
    o_ref[...] = (
        jnp.dot(a_ref[...], w_ref[...], preferred_element_type=jnp.float32)
        + b_ref[...]
    )


def _afh(atom_features, W_cf, b_cf2):
    return pl.pallas_call(
        _afh_body,
        grid=(N_NODES // NODE_BLK,),
        in_specs=[
            pl.BlockSpec((NODE_BLK, N_EMBEDDING), lambda i: (i, 0)),
            pl.BlockSpec((N_EMBEDDING, N_HIDDEN), lambda i: (0, 0)),
            pl.BlockSpec((1, N_HIDDEN), lambda i: (0, 0)),
        ],
        out_specs=pl.BlockSpec((NODE_BLK, N_HIDDEN), lambda i: (i, 0)),
        out_shape=jax.ShapeDtypeStruct((N_ACC, N_HIDDEN), jnp.float32),
    )(atom_features, W_cf, b_cf2)


NBUF = 2


def _gather_body(table_hbm, idx_hbm, out_hbm, idx_v,
                 b0, b1, g0, g1, o0, o1, tab_sh):
    wid = lax.axis_index("s") * NC + lax.axis_index("c")
    s = lax.axis_index("s")
    e0 = wid * W_EDGES
    bufs = (b0, b1)
    gsems = (g0, g1)
    osems = (o0, o1)

    tr = N_ACC // NS
    pltpu.sync_copy(table_hbm.at[pl.ds(s * tr, tr)], tab_sh.at[pl.ds(s * tr, tr)])
    pltpu.sync_copy(idx_hbm.at[pl.ds(wid * W_ROWS, W_ROWS)], idx_v)
    plsc.subcore_barrier()

    def g_copy(t, i):
        return pltpu.make_async_copy(tab_sh.at[idx_v.at[t]], bufs[i], gsems[i])

    def o_copy(t, i):
        return pltpu.make_async_copy(
            bufs[i], out_hbm.at[pl.ds(e0 + t * 128, 128)], osems[i])

    for i in range(NBUF):
        g_copy(i, i).start()
    g_copy(0, 0).wait()
    o_copy(0, 0).start()

    def body(k, carry):
        for i in range(NBUF):
            t = k * NBUF + i
            j = (i + 1) % NBUF
            o_copy(t - NBUF, i).wait()
            g_copy(t, i).start()
            g_copy(t - (NBUF - 1), j).wait()
            o_copy(t - (NBUF - 1), j).start()
        return carry

    lax.fori_loop(1, W_ROWS // NBUF, body, 0)

    for t in range(W_ROWS - (NBUF - 1), W_ROWS):
        i = t % NBUF
        g_copy(t, i).wait()
        o_copy(t, i).start()
    for t in range(W_ROWS - NBUF, W_ROWS):
        o_copy(t, t % NBUF).wait()


@functools.cache
def _gather():
    return functools.partial(
        pl.kernel,
        mesh=plsc.VectorSubcoreMesh(
            core_axis_name="c", subcore_axis_name="s",
            num_cores=NC, num_subcores=NS),
        out_type=jax.ShapeDtypeStruct((PAD_C, N_HIDDEN), jnp.float32),
        scratch_types=(
            [pltpu.VMEM((W_ROWS, 128), jnp.int32)]
            + [pltpu.VMEM((128, N_HIDDEN), jnp.float32)] * NBUF
            + [pltpu.SemaphoreType.DMA] * (2 * NBUF)
            + [pltpu.VMEM_SHARED((N_ACC, N_HIDDEN), jnp.float32)]
        ),
    )(_gather_body)


def _msg_body(dt_ref, g_ref, wdf_ref, bdf_ref, wfc_ref, o_ref):
    dh = (
        lax.dot_general(dt_ref[...], wdf_ref[...],
                        dimension_numbers=(((0,), (0,)), ((), ())),
                        preferred_element_type=jnp.float32)
        + bdf_ref[...]
    )
    o_ref[...] = jnp.tanh(
        jnp.dot(dh * g_ref[...], wfc_ref[...],
                preferred_element_type=jnp.float32)
    )


def _msg(dist_t, g, W_df, b_df2, W_fc, ch):
    return pl.pallas_call(
        _msg_body,
        grid=(BLKS_C,),
        in_specs=[
            pl.BlockSpec((N_DISTANCE, EDGE_BLK), lambda i, ch=ch: (0, i + ch * BLKS_C)),
            pl.BlockSpec((EDGE_BLK, N_HIDDEN), lambda i: (i, 0)),
            pl.BlockSpec((N_DISTANCE, N_HIDDEN), lambda i: (0, 0)),
            pl.BlockSpec((1, N_HIDDEN), lambda i: (0, 0)),
            pl.BlockSpec((N_HIDDEN, N_EMBEDDING), lambda i: (0, 0)),
        ],
        out_specs=pl.BlockSpec((EDGE_BLK, N_EMBEDDING), lambda i: (i, 0)),
        out_shape=jax.ShapeDtypeStruct((PAD_C, N_EMBEDDING), jnp.float32),
    )(dist_t, g, W_df, b_df2, W_fc)


NBUF_S = 2


def _scatter_body(msg_hbm, idx_hbm, p0_hbm, p1_hbm, idx_v,
                  b0, b1, g0, g1, o0, o1, acc_sh):
    c = lax.axis_index("c")
    s = lax.axis_index("s")
    wid = s * NC + c
    e0 = wid * W_EDGES
    bufs = (b0, b1)
    csems = (g0, g1)
    ssems = (o0, o1)

    def zbody(r, carry):
        for l in range(N_HIDDEN // 16):
            b0[r, pl.ds(l * 16, 16)] = jnp.zeros((16,), jnp.float32)
        return carry

    lax.fori_loop(0, 128, zbody, 0)
    for k in range(N_ACC // NS // 128):
        pltpu.sync_copy(b0, acc_sh.at[pl.ds(s * (N_ACC // NS) + k * 128, 128)])
    plsc.subcore_barrier()

    pltpu.sync_copy(idx_hbm.at[pl.ds(wid * W_ROWS, W_ROWS)], idx_v)

    def c_copy(t, i):
        return pltpu.make_async_copy(
            msg_hbm.at[pl.ds(e0 + t * 128, 128)], bufs[i], csems[i])

    def s_copy(t, i):
        return pltpu.make_async_copy(bufs[i], acc_sh.at[idx_v.at[t]], ssems[i])

    for i in range(NBUF_S):
        c_copy(i, i).start()
    c_copy(0, 0).wait()
    s_copy(0, 0).start(add=True)

    def body(k, carry):
        for i in range(NBUF_S):
            t = k * NBUF_S + i
            j = (i + 1) % NBUF_S
            s_copy(t - NBUF_S, i).wait()
            c_copy(t, i).start()
            c_copy(t - (NBUF_S - 1), j).wait()
            s_copy(t - (NBUF_S - 1), j).start(add=True)
        return carry

    lax.fori_loop(1, W_ROWS // NBUF_S, body, 0)

    for t in range(W_ROWS - (NBUF_S - 1), W_ROWS):
        i = t % NBUF_S
        c_copy(t, i).wait()
        s_copy(t, i).start(add=True)
    for t in range(W_ROWS - NBUF_S, W_ROWS):
        s_copy(t, t % NBUF_S).wait()
    plsc.subcore_barrier()

    rows = N_ACC // NS
    r0 = s * rows

    @pl.when(c == 0)
    def _():
        pltpu.sync_copy(acc_sh.at[pl.ds(r0, rows)], p0_hbm.at[pl.ds(r0, rows)])

    @pl.when(c == 1)
    def _():
        pltpu.sync_copy(acc_sh.at[pl.ds(r0, rows)], p1_hbm.at[pl.ds(r0, rows)])


@functools.cache
def _scatter():
    return functools.partial(
        pl.kernel,
        mesh=plsc.VectorSubcoreMesh(
            core_axis_name="c", subcore_axis_name="s",
            num_cores=NC, num_subcores=NS),
        out_type=(
            jax.ShapeDtypeStruct((N_ACC, N_EMBEDDING), jnp.float32),
            jax.ShapeDtypeStruct((N_ACC, N_EMBEDDING), jnp.float32),
        ),
        scratch_types=(
            [pltpu.VMEM((W_ROWS, 128), jnp.int32)]
            + [pltpu.VMEM((128, N_EMBEDDING), jnp.float32)] * NBUF_S
            + [pltpu.SemaphoreType.DMA] * (2 * NBUF_S)
            + [pltpu.VMEM_SHARED((N_ACC, N_EMBEDDING), jnp.float32)]
        ),
    )(_scatter_body)


def _final_body(*refs):
    parts = refs[: 2 * NCH]
    a_ref, afh_ref, bdf_ref, wfc_ref, o_ref = refs[2 * NCH:]
    oii = jnp.tanh(
        jnp.dot(
            bdf_ref[...] * afh_ref[...],
            wfc_ref[...],
            preferred_element_type=jnp.float32,
        )
    )
    acc = a_ref[...] - oii
    for p in parts:
        acc = acc + p[...]
    o_ref[...] = acc


def _final(parts, atom_features, afh, b_df2, W_fc):
    blk = pl.BlockSpec((NODE_BLK, N_EMBEDDING), lambda i: (i, 0))
    return pl.pallas_call(
        _final_body,
        grid=(N_NODES // NODE_BLK,),
        in_specs=(
            [blk] * (2 * NCH)
            + [
                blk,
                pl.BlockSpec((NODE_BLK, N_HIDDEN), lambda i: (i, 0)),
                pl.BlockSpec((1, N_HIDDEN), lambda i: (0, 0)),
                pl.BlockSpec((N_HIDDEN, N_EMBEDDING), lambda i: (0, 0)),
            ]
        ),
        out_specs=pl.BlockSpec((NODE_BLK, N_EMBEDDING), lambda i: (i, 0)),
        out_shape=jax.ShapeDtypeStruct((N_NODES, N_EMBEDDING), jnp.float32),
    )(*parts, atom_features, afh, b_df2, W_fc)


def kernel(atom_features, distance, distance_membership_i, distance_membership_j,
           W_cf, W_df, W_fc, b_cf, b_df):
    mi = distance_membership_i.astype(jnp.int32).reshape(NCH, REAL_C)
    mj = distance_membership_j.astype(jnp.int32).reshape(NCH, REAL_C)
    pad = PAD_C - REAL_C
    mj_pad = jnp.concatenate(
        [mj, jnp.zeros((NCH, pad), jnp.int32)], axis=1
    ).reshape(NCH, CROWS, 128)
    mi_pad = jnp.concatenate(
        [mi, jnp.full((NCH, pad), DUMMY_ROW, jnp.int32)], axis=1
    ).reshape(NCH, CROWS, 128)
    b_cf2 = b_cf.reshape(1, N_HIDDEN)
    b_df2 = b_df.reshape(1, N_HIDDEN)
    dist_t = distance.T

    afh = _afh(atom_features, W_cf, b_cf2)
    parts = []
    for ch in range(NCH):
        g = _gather()(afh, mj_pad[ch])
        msg = _msg(dist_t, g, W_df, b_df2, W_fc, ch)
        p0, p1 = _scatter()(msg, mi_pad[ch])
        parts += [p0, p1]
    return _final(parts, atom_features, afh, b_df2, W_fc)


# speedup vs baseline: 1.1952x; 1.1952x over previous
import functools

import jax
import jax.numpy as jnp
from jax import lax
from jax.experimental import pallas as pl
from jax.experimental.pallas import tpu as pltpu
from jax.experimental.pallas import tpu_sc as plsc

N_NODES = 10000
N_EDGES = 320000
N_EMBEDDING = 128
N_DISTANCE = 100
N_HIDDEN = 128

NC = 2
NS = 16
NW = NC * NS

NCH = 2
EDGE_BLK = 2560
W_ROWS = 40
W_EDGES = W_ROWS * 128
PAD_C = NW * W_EDGES
CROWS = PAD_C // 128
CHUNK_BLKS = (64, 61)

N_ACC = 10240
DUMMY_ROW = N_ACC - 1

NODE_BLK = 2000


def _afh_body(a_ref, w_ref, b_ref, o_ref):
    o_ref[...] = (
        jnp.dot(a_ref[...], w_ref[...], preferred_element_type=jnp.float32)
        + b_ref[...]
    )


def _afh(atom_features, W_cf, b_cf2):
    return pl.pallas_call(
        _afh_body,
        grid=(N_NODES // NODE_BLK,),
        in_specs=[
            pl.BlockSpec((NODE_BLK, N_EMBEDDING), lambda i: (i, 0)),
            pl.BlockSpec((N_EMBEDDING, N_HIDDEN), lambda i: (0, 0)),
            pl.BlockSpec((1, N_HIDDEN), lambda i: (0, 0)),
        ],
        out_specs=pl.BlockSpec((NODE_BLK, N_HIDDEN), lambda i: (i, 0)),
        out_shape=jax.ShapeDtypeStruct((N_ACC, N_HIDDEN), jnp.float32),
    )(atom_features, W_cf, b_cf2)


NBUF = 2


def _gather_body(table_hbm, idx_hbm, out_hbm, idx_v,
                 b0, b1, g0, g1, o0, o1, tab_sh):
    wid = lax.axis_index("s") * NC + lax.axis_index("c")
    s = lax.axis_index("s")
    e0 = wid * W_EDGES
    bufs = (b0, b1)
    gsems = (g0, g1)
    osems = (o0, o1)

    tr = N_ACC // NS
    pltpu.sync_copy(table_hbm.at[pl.ds(s * tr, tr)], tab_sh.at[pl.ds(s * tr, tr)])
    pltpu.sync_copy(idx_hbm.at[pl.ds(wid * W_ROWS, W_ROWS)], idx_v)
    plsc.subcore_barrier()

    def g_copy(t, i):
        return pltpu.make_async_copy(tab_sh.at[idx_v.at[t]], bufs[i], gsems[i])

    def o_copy(t, i):
        return pltpu.make_async_copy(
            bufs[i], out_hbm.at[pl.ds(e0 + t * 128, 128)], osems[i])

    for i in range(NBUF):
        g_copy(i, i).start()
    g_copy(0, 0).wait()
    o_copy(0, 0).start()

    def body(k, carry):
        for i in range(NBUF):
            t = k * NBUF + i
            j = (i + 1) % NBUF
            o_copy(t - NBUF, i).wait()
            g_copy(t, i).start()
            g_copy(t - (NBUF - 1), j).wait()
            o_copy(t - (NBUF - 1), j).start()
        return carry

    lax.fori_loop(1, W_ROWS // NBUF, body, 0)

    for t in range(W_ROWS - (NBUF - 1), W_ROWS):
        i = t % NBUF
        g_copy(t, i).wait()
        o_copy(t, i).start()
    for t in range(W_ROWS - NBUF, W_ROWS):
        o_copy(t, t % NBUF).wait()


@functools.cache
def _gather():
    return functools.partial(
        pl.kernel,
        mesh=plsc.VectorSubcoreMesh(
            core_axis_name="c", subcore_axis_name="s",
            num_cores=NC, num_subcores=NS),
        out_type=jax.ShapeDtypeStruct((PAD_C, N_HIDDEN), jnp.float32),
        scratch_types=(
            [pltpu.VMEM((W_ROWS, 128), jnp.int32)]
            + [pltpu.VMEM((128, N_HIDDEN), jnp.float32)] * NBUF
            + [pltpu.SemaphoreType.DMA] * (2 * NBUF)
            + [pltpu.VMEM_SHARED((N_ACC, N_HIDDEN), jnp.float32)]
        ),
    )(_gather_body)


def _msg_body(dt_ref, g_ref, wdf_ref, bdf_ref, wfc_ref, o_ref):
    dh = (
        lax.dot_general(dt_ref[...], wdf_ref[...],
                        dimension_numbers=(((0,), (0,)), ((), ())),
                        preferred_element_type=jnp.float32)
        + bdf_ref[...]
    )
    o_ref[...] = jnp.tanh(
        jnp.dot(dh * g_ref[...], wfc_ref[...],
                preferred_element_type=jnp.float32)
    )


def _msg(dist_t, g, W_df, b_df2, W_fc, ch):
    blk0 = sum(CHUNK_BLKS[:ch])
    return pl.pallas_call(
        _msg_body,
        grid=(CHUNK_BLKS[ch],),
        in_specs=[
            pl.BlockSpec((N_DISTANCE, EDGE_BLK), lambda i, blk0=blk0: (0, i + blk0)),
            pl.BlockSpec((EDGE_BLK, N_HIDDEN), lambda i: (i, 0)),
            pl.BlockSpec((N_DISTANCE, N_HIDDEN), lambda i: (0, 0)),
            pl.BlockSpec((1, N_HIDDEN), lambda i: (0, 0)),
            pl.BlockSpec((N_HIDDEN, N_EMBEDDING), lambda i: (0, 0)),
        ],
        out_specs=pl.BlockSpec((EDGE_BLK, N_EMBEDDING), lambda i: (i, 0)),
        out_shape=jax.ShapeDtypeStruct((PAD_C, N_EMBEDDING), jnp.float32),
    )(dist_t, g, W_df, b_df2, W_fc)


NBUF_S = 2


def _scatter_body(msg_hbm, idx_hbm, p0_hbm, p1_hbm, idx_v,
                  b0, b1, g0, g1, o0, o1, acc_sh):
    c = lax.axis_index("c")
    s = lax.axis_index("s")
    wid = s * NC + c
    e0 = wid * W_EDGES
    bufs = (b0, b1)
    csems = (g0, g1)
    ssems = (o0, o1)

    def zbody(r, carry):
        for l in range(N_HIDDEN // 16):
            b0[r, pl.ds(l * 16, 16)] = jnp.zeros((16,), jnp.float32)
        return carry

    lax.fori_loop(0, 128, zbody, 0)
    for k in range(N_ACC // NS // 128):
        pltpu.sync_copy(b0, acc_sh.at[pl.ds(s * (N_ACC // NS) + k * 128, 128)])
    plsc.subcore_barrier()

    pltpu.sync_copy(idx_hbm.at[pl.ds(wid * W_ROWS, W_ROWS)], idx_v)

    def c_copy(t, i):
        return pltpu.make_async_copy(
            msg_hbm.at[pl.ds(e0 + t * 128, 128)], bufs[i], csems[i])

    def s_copy(t, i):
        return pltpu.make_async_copy(bufs[i], acc_sh.at[idx_v.at[t]], ssems[i])

    for i in range(NBUF_S):
        c_copy(i, i).start()
    c_copy(0, 0).wait()
    s_copy(0, 0).start(add=True)

    def body(k, carry):
        for i in range(NBUF_S):
            t = k * NBUF_S + i
            j = (i + 1) % NBUF_S
            s_copy(t - NBUF_S, i).wait()
            c_copy(t, i).start()
            c_copy(t - (NBUF_S - 1), j).wait()
            s_copy(t - (NBUF_S - 1), j).start(add=True)
        return carry

    lax.fori_loop(1, W_ROWS // NBUF_S, body, 0)

    for t in range(W_ROWS - (NBUF_S - 1), W_ROWS):
        i = t % NBUF_S
        c_copy(t, i).wait()
        s_copy(t, i).start(add=True)
    for t in range(W_ROWS - NBUF_S, W_ROWS):
        s_copy(t, t % NBUF_S).wait()
    plsc.subcore_barrier()

    rows = N_ACC // NS
    r0 = s * rows

    @pl.when(c == 0)
    def _():
        pltpu.sync_copy(acc_sh.at[pl.ds(r0, rows)], p0_hbm.at[pl.ds(r0, rows)])

    @pl.when(c == 1)
    def _():
        pltpu.sync_copy(acc_sh.at[pl.ds(r0, rows)], p1_hbm.at[pl.ds(r0, rows)])


@functools.cache
def _scatter():
    return functools.partial(
        pl.kernel,
        mesh=plsc.VectorSubcoreMesh(
            core_axis_name="c", subcore_axis_name="s",
            num_cores=NC, num_subcores=NS),
        out_type=(
            jax.ShapeDtypeStruct((N_ACC, N_EMBEDDING), jnp.float32),
            jax.ShapeDtypeStruct((N_ACC, N_EMBEDDING), jnp.float32),
        ),
        scratch_types=(
            [pltpu.VMEM((W_ROWS, 128), jnp.int32)]
            + [pltpu.VMEM((128, N_EMBEDDING), jnp.float32)] * NBUF_S
            + [pltpu.SemaphoreType.DMA] * (2 * NBUF_S)
            + [pltpu.VMEM_SHARED((N_ACC, N_EMBEDDING), jnp.float32)]
        ),
    )(_scatter_body)


def _final_body(*refs):
    parts = refs[: 2 * NCH]
    a_ref, afh_ref, bdf_ref, wfc_ref, o_ref = refs[2 * NCH:]
    oii = jnp.tanh(
        jnp.dot(
            bdf_ref[...] * afh_ref[...],
            wfc_ref[...],
            preferred_element_type=jnp.float32,
        )
    )
    acc = a_ref[...] - oii
    for p in parts:
        acc = acc + p[...]
    o_ref[...] = acc


def _final(parts, atom_features, afh, b_df2, W_fc):
    blk = pl.BlockSpec((NODE_BLK, N_EMBEDDING), lambda i: (i, 0))
    return pl.pallas_call(
        _final_body,
        grid=(N_NODES // NODE_BLK,),
        in_specs=(
            [blk] * (2 * NCH)
            + [
                blk,
                pl.BlockSpec((NODE_BLK, N_HIDDEN), lambda i: (i, 0)),
                pl.BlockSpec((1, N_HIDDEN), lambda i: (0, 0)),
                pl.BlockSpec((N_HIDDEN, N_EMBEDDING), lambda i: (0, 0)),
            ]
        ),
        out_specs=pl.BlockSpec((NODE_BLK, N_EMBEDDING), lambda i: (i, 0)),
        out_shape=jax.ShapeDtypeStruct((N_NODES, N_EMBEDDING), jnp.float32),
    )(*parts, atom_features, afh, b_df2, W_fc)


def kernel(atom_features, distance, distance_membership_i, distance_membership_j,
           W_cf, W_df, W_fc, b_cf, b_df):
    mi = distance_membership_i.astype(jnp.int32)
    mj = distance_membership_j.astype(jnp.int32)
    r0 = CHUNK_BLKS[0] * EDGE_BLK
    pad1 = PAD_C - (N_EDGES - r0)
    mj_pad = [
        mj[:r0].reshape(CROWS, 128),
        jnp.concatenate([mj[r0:], jnp.zeros((pad1,), jnp.int32)]).reshape(CROWS, 128),
    ]
    mi_pad = [
        mi[:r0].reshape(CROWS, 128),
        jnp.concatenate(
            [mi[r0:], jnp.full((pad1,), DUMMY_ROW, jnp.int32)]
        ).reshape(CROWS, 128),
    ]
    b_cf2 = b_cf.reshape(1, N_HIDDEN)
    b_df2 = b_df.reshape(1, N_HIDDEN)
    dist_t = distance.T

    afh = _afh(atom_features, W_cf, b_cf2)
    parts = []
    for ch in range(NCH):
        g = _gather()(afh, mj_pad[ch])
        msg = _msg(dist_t, g, W_df, b_df2, W_fc, ch)
        p0, p1 = _scatter()(msg, mi_pad[ch])
        parts += [p0, p1]
    return _final(parts, atom_features, afh, b_df2, W_fc)


# speedup vs baseline: 1.2232x; 1.0234x over previous
import functools

import jax
import jax.numpy as jnp
from jax import lax
from jax.experimental import pallas as pl
from jax.experimental.pallas import tpu as pltpu
from jax.experimental.pallas import tpu_sc as plsc

N_NODES = 10000
N_EDGES = 320000
N_EMBEDDING = 128
N_DISTANCE = 100
N_HIDDEN = 128

NC = 2
NS = 16
NW = NC * NS

EDGE_BLK = 2560
CHUNK_BLKS = (32, 52, 41)
CHUNK_ROWS = (20, 34, 26)
NCH = len(CHUNK_BLKS)

N_ACC = 10240
DUMMY_ROW = N_ACC - 1

NODE_BLK = 2000


def _afh_body(a_ref, w_ref, b_ref, o_ref):
    o_ref[...] = (
        jnp.dot(a_ref[...], w_ref[...], preferred_element_type=jnp.float32)
        + b_ref[...]
    )


def _afh(atom_features, W_cf, b_cf2):
    return pl.pallas_call(
        _afh_body,
        grid=(N_NODES // NODE_BLK,),
        in_specs=[
            pl.BlockSpec((NODE_BLK, N_EMBEDDING), lambda i: (i, 0)),
            pl.BlockSpec((N_EMBEDDING, N_HIDDEN), lambda i: (0, 0)),
            pl.BlockSpec((1, N_HIDDEN), lambda i: (0, 0)),
        ],
        out_specs=pl.BlockSpec((NODE_BLK, N_HIDDEN), lambda i: (i, 0)),
        out_shape=jax.ShapeDtypeStruct((N_ACC, N_HIDDEN), jnp.float32),
    )(atom_features, W_cf, b_cf2)


NBUF = 2


@functools.cache
def _gather(rows):
    def body_fn(table_hbm, idx_hbm, out_hbm, idx_v,
                b0, b1, g0, g1, o0, o1, tab_sh):
        wid = lax.axis_index("s") * NC + lax.axis_index("c")
        s = lax.axis_index("s")
        e0 = wid * rows * 128
        bufs = (b0, b1)
        gsems = (g0, g1)
        osems = (o0, o1)

        tr = N_ACC // NS
        pltpu.sync_copy(table_hbm.at[pl.ds(s * tr, tr)],
                        tab_sh.at[pl.ds(s * tr, tr)])
        pltpu.sync_copy(idx_hbm.at[wid], idx_v)
        plsc.subcore_barrier()

        def g_copy(t, i):
            return pltpu.make_async_copy(tab_sh.at[idx_v.at[t]], bufs[i], gsems[i])

        def o_copy(t, i):
            return pltpu.make_async_copy(
                bufs[i], out_hbm.at[pl.ds(e0 + t * 128, 128)], osems[i])

        for i in range(NBUF):
            g_copy(i, i).start()
        g_copy(0, 0).wait()
        o_copy(0, 0).start()

        def body(k, carry):
            for i in range(NBUF):
                t = k * NBUF + i
                j = (i + 1) % NBUF
                o_copy(t - NBUF, i).wait()
                g_copy(t, i).start()
                g_copy(t - (NBUF - 1), j).wait()
                o_copy(t - (NBUF - 1), j).start()
            return carry

        lax.fori_loop(1, rows // NBUF, body, 0)

        for t in range(rows - (NBUF - 1), rows):
            i = t % NBUF
            g_copy(t, i).wait()
            o_copy(t, i).start()
        for t in range(rows - NBUF, rows):
            o_copy(t, t % NBUF).wait()

    return functools.partial(
        pl.kernel,
        mesh=plsc.VectorSubcoreMesh(
            core_axis_name="c", subcore_axis_name="s",
            num_cores=NC, num_subcores=NS),
        out_type=jax.ShapeDtypeStruct((NW * rows * 128, N_HIDDEN), jnp.float32),
        scratch_types=(
            [pltpu.VMEM((rows, 128), jnp.int32)]
            + [pltpu.VMEM((128, N_HIDDEN), jnp.float32)] * NBUF
            + [pltpu.SemaphoreType.DMA] * (2 * NBUF)
            + [pltpu.VMEM_SHARED((N_ACC, N_HIDDEN), jnp.float32)]
        ),
    )(body_fn)


def _msg_body(dt_ref, g_ref, wdf_ref, bdf_ref, wfc_ref, o_ref):
    dh = (
        lax.dot_general(dt_ref[...], wdf_ref[...],
                        dimension_numbers=(((0,), (0,)), ((), ())),
                        preferred_element_type=jnp.float32)
        + bdf_ref[...]
    )
    o_ref[...] = jnp.tanh(
        jnp.dot(dh * g_ref[...], wfc_ref[...],
                preferred_element_type=jnp.float32)
    )


def _msg(dist_t, g, W_df, b_df2, W_fc, ch):
    blk0 = sum(CHUNK_BLKS[:ch])
    pad_c = NW * CHUNK_ROWS[ch] * 128
    return pl.pallas_call(
        _msg_body,
        grid=(CHUNK_BLKS[ch],),
        in_specs=[
            pl.BlockSpec((N_DISTANCE, EDGE_BLK), lambda i, blk0=blk0: (0, i + blk0)),
            pl.BlockSpec((EDGE_BLK, N_HIDDEN), lambda i: (i, 0)),
            pl.BlockSpec((N_DISTANCE, N_HIDDEN), lambda i: (0, 0)),
            pl.BlockSpec((1, N_HIDDEN), lambda i: (0, 0)),
            pl.BlockSpec((N_HIDDEN, N_EMBEDDING), lambda i: (0, 0)),
        ],
        out_specs=pl.BlockSpec((EDGE_BLK, N_EMBEDDING), lambda i: (i, 0)),
        out_shape=jax.ShapeDtypeStruct((pad_c, N_EMBEDDING), jnp.float32),
    )(dist_t, g, W_df, b_df2, W_fc)


NBUF_S = 2


@functools.cache
def _scatter(rows):
    def body_fn(msg_hbm, idx_hbm, p0_hbm, p1_hbm, idx_v,
                b0, b1, g0, g1, o0, o1, acc_sh):
        c = lax.axis_index("c")
        s = lax.axis_index("s")
        wid = s * NC + c
        e0 = wid * rows * 128
        bufs = (b0, b1)
        csems = (g0, g1)
        ssems = (o0, o1)

        def zbody(r, carry):
            for l in range(N_HIDDEN // 16):
                b0[r, pl.ds(l * 16, 16)] = jnp.zeros((16,), jnp.float32)
            return carry

        lax.fori_loop(0, 128, zbody, 0)
        for k in range(N_ACC // NS // 128):
            pltpu.sync_copy(b0, acc_sh.at[pl.ds(s * (N_ACC // NS) + k * 128, 128)])
        plsc.subcore_barrier()

        pltpu.sync_copy(idx_hbm.at[wid], idx_v)

        def c_copy(t, i):
            return pltpu.make_async_copy(
                msg_hbm.at[pl.ds(e0 + t * 128, 128)], bufs[i], csems[i])

        def s_copy(t, i):
            return pltpu.make_async_copy(bufs[i], acc_sh.at[idx_v.at[t]], ssems[i])

        for i in range(NBUF_S):
            c_copy(i, i).start()
        c_copy(0, 0).wait()
        s_copy(0, 0).start(add=True)

        def body(k, carry):
            for i in range(NBUF_S):
                t = k * NBUF_S + i
                j = (i + 1) % NBUF_S
                s_copy(t - NBUF_S, i).wait()
                c_copy(t, i).start()
                c_copy(t - (NBUF_S - 1), j).wait()
                s_copy(t - (NBUF_S - 1), j).start(add=True)
            return carry

        lax.fori_loop(1, rows // NBUF_S, body, 0)

        for t in range(rows - (NBUF_S - 1), rows):
            i = t % NBUF_S
            c_copy(t, i).wait()
            s_copy(t, i).start(add=True)
        for t in range(rows - NBUF_S, rows):
            s_copy(t, t % NBUF_S).wait()
        plsc.subcore_barrier()

        orows = N_ACC // NS
        r0 = s * orows

        @pl.when(c == 0)
        def _():
            pltpu.sync_copy(acc_sh.at[pl.ds(r0, orows)], p0_hbm.at[pl.ds(r0, orows)])

        @pl.when(c == 1)
        def _():
            pltpu.sync_copy(acc_sh.at[pl.ds(r0, orows)], p1_hbm.at[pl.ds(r0, orows)])

    return functools.partial(
        pl.kernel,
        mesh=plsc.VectorSubcoreMesh(
            core_axis_name="c", subcore_axis_name="s",
            num_cores=NC, num_subcores=NS),
        out_type=(
            jax.ShapeDtypeStruct((N_ACC, N_EMBEDDING), jnp.float32),
            jax.ShapeDtypeStruct((N_ACC, N_EMBEDDING), jnp.float32),
        ),
        scratch_types=(
            [pltpu.VMEM((rows, 128), jnp.int32)]
            + [pltpu.VMEM((128, N_EMBEDDING), jnp.float32)] * NBUF_S
            + [pltpu.SemaphoreType.DMA] * (2 * NBUF_S)
            + [pltpu.VMEM_SHARED((N_ACC, N_EMBEDDING), jnp.float32)]
        ),
    )(body_fn)


def _final_body(*refs):
    parts = refs[: 2 * NCH]
    a_ref, afh_ref, bdf_ref, wfc_ref, o_ref = refs[2 * NCH:]
    oii = jnp.tanh(
        jnp.dot(
            bdf_ref[...] * afh_ref[...],
            wfc_ref[...],
            preferred_element_type=jnp.float32,
        )
    )
    acc = a_ref[...] - oii
    for p in parts:
        acc = acc + p[...]
    o_ref[...] = acc


def _final(parts, atom_features, afh, b_df2, W_fc):
    blk = pl.BlockSpec((NODE_BLK, N_EMBEDDING), lambda i: (i, 0))
    return pl.pallas_call(
        _final_body,
        grid=(N_NODES // NODE_BLK,),
        in_specs=(
            [blk] * (2 * NCH)
            + [
                blk,
                pl.BlockSpec((NODE_BLK, N_HIDDEN), lambda i: (i, 0)),
                pl.BlockSpec((1, N_HIDDEN), lambda i: (0, 0)),
                pl.BlockSpec((N_HIDDEN, N_EMBEDDING), lambda i: (0, 0)),
            ]
        ),
        out_specs=pl.BlockSpec((NODE_BLK, N_EMBEDDING), lambda i: (i, 0)),
        out_shape=jax.ShapeDtypeStruct((N_NODES, N_EMBEDDING), jnp.float32),
    )(*parts, atom_features, afh, b_df2, W_fc)


def kernel(atom_features, distance, distance_membership_i, distance_membership_j,
           W_cf, W_df, W_fc, b_cf, b_df):
    mi = distance_membership_i.astype(jnp.int32)
    mj = distance_membership_j.astype(jnp.int32)
    mj_pad, mi_pad = [], []
    off = 0
    for ch in range(NCH):
        real = CHUNK_BLKS[ch] * EDGE_BLK
        rows = CHUNK_ROWS[ch]
        pad_c = NW * rows * 128
        mj_pad.append(jnp.concatenate(
            [mj[off:off + real], jnp.zeros((pad_c - real,), jnp.int32)]
        ).reshape(NW, rows, 128))
        mi_pad.append(jnp.concatenate(
            [mi[off:off + real], jnp.full((pad_c - real,), DUMMY_ROW, jnp.int32)]
        ).reshape(NW, rows, 128))
        off += real
    b_cf2 = b_cf.reshape(1, N_HIDDEN)
    b_df2 = b_df.reshape(1, N_HIDDEN)
    dist_t = distance.T

    afh = _afh(atom_features, W_cf, b_cf2)
    parts = []
    for ch in range(NCH):
        rows = CHUNK_ROWS[ch]
        g = _gather(rows)(afh, mj_pad[ch])
        msg = _msg(dist_t, g, W_df, b_df2, W_fc, ch)
        p0, p1 = _scatter(rows)(msg, mi_pad[ch])
        parts += [p0, p1]
    return _final(parts, atom_features, afh, b_df2, W_fc)
